# Initial kernel scaffold; baseline (speedup 1.0000x reference)
import jax
import jax.numpy as jnp
from jax.experimental import pallas as pl


def kernel(residue_types, table):
    raise NotImplementedError("write your pallas kernel here")


# speedup vs baseline: 1.3554x; 1.3554x over previous
import functools

import jax
import jax.numpy as jnp
from jax import lax
from jax.experimental import pallas as pl
from jax.experimental.pallas import tpu as pltpu
from jax.experimental.pallas import tpu_sc as plsc

_NC = 2
_NS = 16
_NW = _NC * _NS
_D = 80
_C = 512
_RJ = _C // 128


@functools.partial(jax.jit, static_argnames=("B",))
def _sc_embed(idx2d, table, B):
    b_per_w = B // _NW
    n_chunks = b_per_w // _C
    assert b_per_w % _C == 0

    mesh = plsc.VectorSubcoreMesh(core_axis_name="c", subcore_axis_name="s")

    @functools.partial(
        pl.kernel,
        mesh=mesh,
        out_type=jax.ShapeDtypeStruct((B, _D), jnp.float32),
        compiler_params=pltpu.CompilerParams(use_tc_tiling_on_sc=False),
        scratch_types=[
            pltpu.VMEM((_RJ, 128), jnp.int32),
            pltpu.VMEM((_C, _D), jnp.float32),
            pltpu.SemaphoreType.DMA,
        ],
    )
    def k(idx_hbm, table_hbm, out_hbm, idx_v, rows_v, sg):
        wid = lax.axis_index("s") * _NC + lax.axis_index("c")
        wrow0 = wid * (b_per_w // 128)
        wbase = wid * b_per_w

        def body(g, carry):
            irow = wrow0 + g * _RJ
            base = wbase + g * _C
            pltpu.sync_copy(idx_hbm.at[pl.ds(irow, _RJ)], idx_v)
            cps = [
                pltpu.async_copy(
                    table_hbm.at[idx_v.at[j]],
                    rows_v.at[pl.ds(j * 128, 128)],
                    sg,
                )
                for j in range(_RJ)
            ]
            for cp in cps:
                cp.wait()
            pltpu.sync_copy(rows_v, out_hbm.at[pl.ds(base, _C)])
            return carry

        lax.fori_loop(0, n_chunks, body, 0)

    return k(idx2d, table)


def kernel(residue_types, table):
    S, T = residue_types.shape
    B = S * T
    idx2d = residue_types.reshape(B // 128, 128)
    out = _sc_embed(idx2d, table, B)
    return out.reshape(S, T, _D)


# speedup vs baseline: 3.2052x; 2.3648x over previous
import functools

import jax
import jax.numpy as jnp
from jax import lax
from jax.experimental import pallas as pl
from jax.experimental.pallas import tpu as pltpu
from jax.experimental.pallas import tpu_sc as plsc

_NC = 2
_NS = 16
_NW = _NC * _NS
_D = 80
_C = 512
_RJ = _C // 128


@functools.partial(jax.jit, static_argnames=("B",))
def _sc_embed(idx2d, table, B):
    b_per_w = B // _NW
    n_chunks = b_per_w // _C
    assert b_per_w % _C == 0

    mesh = plsc.VectorSubcoreMesh(core_axis_name="c", subcore_axis_name="s")

    @functools.partial(
        pl.kernel,
        mesh=mesh,
        out_type=jax.ShapeDtypeStruct((B, _D), jnp.float32),
        compiler_params=pltpu.CompilerParams(use_tc_tiling_on_sc=False),
        scratch_types=[
            pltpu.VMEM((_RJ, 128), jnp.int32),
            pltpu.VMEM((_C, _D), jnp.float32),
            pltpu.SemaphoreType.DMA,
        ],
    )
    def k(idx_hbm, table_hbm, out_hbm, idx_v, rows_v, sg):
        wid = lax.axis_index("s") * _NC + lax.axis_index("c")
        wrow0 = wid * (b_per_w // 128)
        wbase = wid * b_per_w
        off = wid * 21

        def body(g, carry):
            irow = wrow0 + g * _RJ
            base = wbase + g * _C
            pltpu.sync_copy(idx_hbm.at[pl.ds(irow, _RJ)], idx_v)
            for j in range(_RJ):
                for q in range(128 // 16):
                    sl = idx_v.at[j][pl.ds(q * 16, 16)]
                    idx_v.at[j][pl.ds(q * 16, 16)] = sl + off
            cps = [
                pltpu.async_copy(
                    table_hbm.at[idx_v.at[j]],
                    rows_v.at[pl.ds(j * 128, 128)],
                    sg,
                )
                for j in range(_RJ)
            ]
            for cp in cps:
                cp.wait()
            pltpu.sync_copy(rows_v, out_hbm.at[pl.ds(base, _C)])
            return carry

        lax.fori_loop(0, n_chunks, body, 0)

    return k(idx2d, table)


def kernel(residue_types, table):
    S, T = residue_types.shape
    B = S * T
    idx2d = residue_types.reshape(B // 128, 128)
    table_rep = jnp.tile(table, (_NW, 1))
    out = _sc_embed(idx2d, table_rep, B)
    return out.reshape(S, T, _D)


# speedup vs baseline: 3.2466x; 1.0129x over previous
import functools

import jax
import jax.numpy as jnp
from jax import lax
from jax.experimental import pallas as pl
from jax.experimental.pallas import tpu as pltpu
from jax.experimental.pallas import tpu_sc as plsc

_NC = 2
_NS = 16
_NW = _NC * _NS
_D = 80
_C = 512
_RJ = _C // 128


@functools.partial(jax.jit, static_argnames=("B",))
def _sc_embed(idx2d, table, B):
    b_per_w = B // _NW
    n_chunks = b_per_w // _C
    assert b_per_w % _C == 0

    mesh = plsc.VectorSubcoreMesh(core_axis_name="c", subcore_axis_name="s")

    @functools.partial(
        pl.kernel,
        mesh=mesh,
        out_type=jax.ShapeDtypeStruct((B, _D), jnp.float32),
        compiler_params=pltpu.CompilerParams(use_tc_tiling_on_sc=False),
        scratch_types=[
            pltpu.VMEM((2, _RJ, 128), jnp.int32),
            pltpu.VMEM((2, _C, _D), jnp.float32),
            pltpu.SemaphoreType.DMA,
            pltpu.SemaphoreType.DMA,
            pltpu.SemaphoreType.DMA,
            pltpu.SemaphoreType.DMA,
        ],
    )
    def k(idx_hbm, table_hbm, out_hbm, idx_v, rows_v, sg0, sg1, ss0, ss1):
        wid = lax.axis_index("s") * _NC + lax.axis_index("c")
        wrow0 = wid * (b_per_w // 128)
        wbase = wid * b_per_w
        off = wid * 21
        npairs = n_chunks // 2

        def stage_idx(g, slot):
            pltpu.sync_copy(idx_hbm.at[pl.ds(wrow0 + g * _RJ, _RJ)], idx_v.at[slot])
            for j in range(_RJ):
                for q in range(128 // 16):
                    sl = idx_v.at[slot, j][pl.ds(q * 16, 16)]
                    idx_v.at[slot, j][pl.ds(q * 16, 16)] = sl + off

        def fire_gathers(slot, sem):
            return [
                pltpu.async_copy(
                    table_hbm.at[idx_v.at[slot, j]],
                    rows_v.at[slot, pl.ds(j * 128, 128)],
                    sem,
                )
                for j in range(_RJ)
            ]

        def fire_scatter(g, slot, sem):
            return pltpu.async_copy(
                rows_v.at[slot], out_hbm.at[pl.ds(wbase + g * _C, _C)], sem
            )

        def drain_odd_scatter():
            pltpu.make_async_copy(
                rows_v.at[1], out_hbm.at[pl.ds(wbase, _C)], ss1
            ).wait()

        def body(p, carry):
            g0 = 2 * p
            stage_idx(g0, 0)
            g_cps = fire_gathers(0, sg0)

            @pl.when(p >= 1)
            def _():
                drain_odd_scatter()

            for cp in g_cps:
                cp.wait()
            sc0 = fire_scatter(g0, 0, ss0)

            stage_idx(g0 + 1, 1)
            for cp in fire_gathers(1, sg1):
                cp.wait()
            sc0.wait()
            fire_scatter(g0 + 1, 1, ss1)
            return carry

        lax.fori_loop(0, npairs, body, 0)
        drain_odd_scatter()

    return k(idx2d, table)


def kernel(residue_types, table):
    S, T = residue_types.shape
    B = S * T
    idx2d = residue_types.reshape(B // 128, 128)
    table_rep = jnp.tile(table, (_NW, 1))
    out = _sc_embed(idx2d, table_rep, B)
    return out.reshape(S, T, _D)


# speedup vs baseline: 4.3071x; 1.3266x over previous
import functools

import jax
import jax.numpy as jnp
from jax import lax
from jax.experimental import pallas as pl
from jax.experimental.pallas import tpu as pltpu
from jax.experimental.pallas import tpu_sc as plsc

_NC = 2
_NS = 16
_NW = _NC * _NS
_D = 80
_DP = 128
_C = 256
_RJ = _C // 128


@functools.partial(jax.jit, static_argnames=("B",))
def _sc_embed(idx2d, table, B):
    b_per_w = B // _NW
    n_chunks = b_per_w // _C
    assert b_per_w % _C == 0 and n_chunks % 2 == 0

    mesh = plsc.VectorSubcoreMesh(core_axis_name="c", subcore_axis_name="s")

    @functools.partial(
        pl.kernel,
        mesh=mesh,
        out_type=jax.ShapeDtypeStruct((B, _DP), jnp.float32),
        scratch_types=[
            pltpu.VMEM((2, _RJ, 128), jnp.int32),
            pltpu.VMEM((2, _C, _DP), jnp.float32),
            pltpu.SemaphoreType.DMA,
            pltpu.SemaphoreType.DMA,
            pltpu.SemaphoreType.DMA,
            pltpu.SemaphoreType.DMA,
        ],
    )
    def k(idx_hbm, table_hbm, out_hbm, idx_v, rows_v, sg0, sg1, ss0, ss1):
        wid = lax.axis_index("s") * _NC + lax.axis_index("c")
        wrow0 = wid * (b_per_w // 128)
        wbase = wid * b_per_w
        off = wid * 21
        npairs = n_chunks // 2

        def stage_idx(g, slot):
            pltpu.sync_copy(idx_hbm.at[pl.ds(wrow0 + g * _RJ, _RJ)], idx_v.at[slot])
            for j in range(_RJ):
                for q in range(128 // 16):
                    sl = idx_v.at[slot, j][pl.ds(q * 16, 16)]
                    idx_v.at[slot, j][pl.ds(q * 16, 16)] = sl + off

        def fire_gathers(slot, sem):
            return [
                pltpu.async_copy(
                    table_hbm.at[idx_v.at[slot, j]],
                    rows_v.at[slot, pl.ds(j * 128, 128)],
                    sem,
                )
                for j in range(_RJ)
            ]

        def fire_scatter(g, slot, sem):
            return pltpu.async_copy(
                rows_v.at[slot],
                out_hbm.at[pl.ds(wbase + g * _C, _C)],
                sem,
            )

        def drain_odd_scatter():
            pltpu.make_async_copy(
                rows_v.at[1], out_hbm.at[pl.ds(wbase, _C)], ss1
            ).wait()

        def body(p, carry):
            g0 = 2 * p
            stage_idx(g0, 0)
            g_cps = fire_gathers(0, sg0)

            @pl.when(p >= 1)
            def _():
                drain_odd_scatter()

            for cp in g_cps:
                cp.wait()
            sc0 = fire_scatter(g0, 0, ss0)

            stage_idx(g0 + 1, 1)
            for cp in fire_gathers(1, sg1):
                cp.wait()
            sc0.wait()
            fire_scatter(g0 + 1, 1, ss1)
            return carry

        lax.fori_loop(0, npairs, body, 0)
        drain_odd_scatter()

    return k(idx2d, table)


def kernel(residue_types, table):
    S, T = residue_types.shape
    B = S * T
    idx2d = residue_types.reshape(B // 128, 128)
    table_rep = jnp.tile(jnp.pad(table, ((0, 0), (0, _DP - _D))), (_NW, 1))
    out = _sc_embed(idx2d, table_rep, B)
    return out[:, :_D].reshape(S, T, _D)
